# baseline (speedup 1.0000x reference)
import functools

import jax
import jax.numpy as jnp
from jax.experimental import pallas as pl
from jax.experimental.pallas import tpu as pltpu

NUM_CODES = 1024
NUM_CHANNELS = 64
BT = 2048


def _vq_logits_kernel(keys_ref, emb_ref, out_ref):
    k = keys_ref[...]
    e = emb_ref[...]
    cross = jax.lax.dot_general(
        k, e, (((1,), (1,)), ((), ())),
        preferred_element_type=jnp.float32,
    )
    k_sq = jnp.sum(k * k, axis=1, keepdims=True)
    e_sq = jnp.sum(e * e, axis=1)
    out_ref[...] = (2.0 * cross - k_sq) - e_sq[None, :]


@jax.jit
def kernel(keys, embeddings):
    n_tokens = keys.shape[0]
    grid = (n_tokens // BT,)
    return pl.pallas_call(
        _vq_logits_kernel,
        grid=grid,
        in_specs=[
            pl.BlockSpec((BT, NUM_CHANNELS), lambda i: (i, 0)),
            pl.BlockSpec((NUM_CODES, NUM_CHANNELS), lambda i: (0, 0)),
        ],
        out_specs=pl.BlockSpec((BT, NUM_CODES), lambda i: (i, 0)),
        out_shape=jax.ShapeDtypeStruct((n_tokens, NUM_CODES), jnp.float32),
        compiler_params=pltpu.CompilerParams(
            dimension_semantics=("arbitrary",),
        ),
    )(keys, embeddings)


# speedup vs baseline: 1.0009x; 1.0009x over previous
import functools

import jax
import jax.numpy as jnp
from jax.experimental import pallas as pl
from jax.experimental.pallas import tpu as pltpu

NUM_CODES = 1024
NUM_CHANNELS = 64
BT = 4096


def _vq_logits_kernel(keys_ref, emb_ref, out_ref):
    k = keys_ref[...]
    e = emb_ref[...]
    cross = jax.lax.dot_general(
        k, e, (((1,), (1,)), ((), ())),
        preferred_element_type=jnp.float32,
    )
    k_sq = jnp.sum(k * k, axis=1, keepdims=True)
    e_sq = jnp.sum(e * e, axis=1)
    out_ref[...] = (2.0 * cross - k_sq) - e_sq[None, :]


@jax.jit
def kernel(keys, embeddings):
    n_tokens = keys.shape[0]
    grid = (n_tokens // BT,)
    return pl.pallas_call(
        _vq_logits_kernel,
        grid=grid,
        in_specs=[
            pl.BlockSpec((BT, NUM_CHANNELS), lambda i: (i, 0)),
            pl.BlockSpec((NUM_CODES, NUM_CHANNELS), lambda i: (0, 0)),
        ],
        out_specs=pl.BlockSpec((BT, NUM_CODES), lambda i: (i, 0)),
        out_shape=jax.ShapeDtypeStruct((n_tokens, NUM_CODES), jnp.float32),
        compiler_params=pltpu.CompilerParams(
            dimension_semantics=("arbitrary",),
        ),
    )(keys, embeddings)


# speedup vs baseline: 1.0062x; 1.0052x over previous
import functools

import jax
import jax.numpy as jnp
from jax.experimental import pallas as pl
from jax.experimental.pallas import tpu as pltpu

NUM_CODES = 1024
NUM_CHANNELS = 64
BT = 2048


def _vq_logits_kernel(keys_ref, emb_ref, out_ref):
    k = keys_ref[...]
    e = emb_ref[...]
    k_sq = jnp.sum(k * k, axis=1, keepdims=True)
    e_sq = jnp.sum(e * e, axis=1, keepdims=True)
    ones_k = jnp.ones_like(k_sq)
    ones_e = jnp.ones_like(e_sq)
    a = jnp.concatenate([k + k, -k_sq, ones_k], axis=1)
    b = jnp.concatenate([e, ones_e, -e_sq], axis=1)
    out_ref[...] = jax.lax.dot_general(
        a, b, (((1,), (1,)), ((), ())),
        preferred_element_type=jnp.float32,
    )


@jax.jit
def kernel(keys, embeddings):
    n_tokens = keys.shape[0]
    grid = (n_tokens // BT,)
    return pl.pallas_call(
        _vq_logits_kernel,
        grid=grid,
        in_specs=[
            pl.BlockSpec((BT, NUM_CHANNELS), lambda i: (i, 0)),
            pl.BlockSpec((NUM_CODES, NUM_CHANNELS), lambda i: (0, 0)),
        ],
        out_specs=pl.BlockSpec((BT, NUM_CODES), lambda i: (i, 0)),
        out_shape=jax.ShapeDtypeStruct((n_tokens, NUM_CODES), jnp.float32),
        compiler_params=pltpu.CompilerParams(
            dimension_semantics=("arbitrary",),
        ),
    )(keys, embeddings)


# speedup vs baseline: 1.0112x; 1.0050x over previous
import functools

import jax
import jax.numpy as jnp
from jax.experimental import pallas as pl
from jax.experimental.pallas import tpu as pltpu

NUM_CODES = 1024
NUM_CHANNELS = 64
BT = 2048
NBUF = 4


def _vq_logits_kernel(n_steps, keys_ref, emb_ref, out_ref, scratch, sems):
    i = pl.program_id(0)
    slot = jax.lax.rem(i, NBUF)

    @pl.when(i >= NBUF)
    def _wait_prev():
        pltpu.make_async_copy(
            scratch.at[slot],
            out_ref.at[pl.ds((i - NBUF) * BT, BT), :],
            sems.at[slot],
        ).wait()

    k = keys_ref[...]
    e = emb_ref[...]
    k_sq = jnp.sum(k * k, axis=1, keepdims=True)
    e_sq = jnp.sum(e * e, axis=1, keepdims=True)
    a = jnp.concatenate([k + k, -k_sq, jnp.ones_like(k_sq)], axis=1)
    b = jnp.concatenate([e, jnp.ones_like(e_sq), -e_sq], axis=1)
    scratch[slot] = jax.lax.dot_general(
        a, b, (((1,), (1,)), ((), ())),
        preferred_element_type=jnp.float32,
    )

    pltpu.make_async_copy(
        scratch.at[slot],
        out_ref.at[pl.ds(i * BT, BT), :],
        sems.at[slot],
    ).start()

    @pl.when(i == n_steps - 1)
    def _drain():
        for s in range(NBUF):
            pltpu.make_async_copy(
                scratch.at[s],
                out_ref.at[pl.ds(0, BT), :],
                sems.at[s],
            ).wait()


@jax.jit
def kernel(keys, embeddings):
    n_tokens = keys.shape[0]
    n_steps = n_tokens // BT
    return pl.pallas_call(
        functools.partial(_vq_logits_kernel, n_steps),
        grid=(n_steps,),
        in_specs=[
            pl.BlockSpec((BT, NUM_CHANNELS), lambda i: (i, 0)),
            pl.BlockSpec((NUM_CODES, NUM_CHANNELS), lambda i: (0, 0)),
        ],
        out_specs=pl.BlockSpec(memory_space=pl.ANY),
        out_shape=jax.ShapeDtypeStruct((n_tokens, NUM_CODES), jnp.float32),
        scratch_shapes=[
            pltpu.VMEM((NBUF, BT, NUM_CODES), jnp.float32),
            pltpu.SemaphoreType.DMA((NBUF,)),
        ],
        compiler_params=pltpu.CompilerParams(
            dimension_semantics=("arbitrary",),
        ),
    )(keys, embeddings)
